# initial kernel scaffold (unmeasured)
import functools

import jax
import jax.numpy as jnp
from jax import lax
from jax.experimental import pallas as pl
from jax.experimental.pallas import tpu as pltpu

N_DEV = 16
M_PER = 512


def kernel(x, w_mat):
    k_glob, k_per = x.shape
    _, n = w_mat.shape

    def body(x_ref, w_ref, out_ref, send_buf, recv_buf,
             send_sem, recv_sem, credit_sem):
        d = lax.axis_index("i")
        left = (d - 1) % N_DEV
        right = (d + 1) % N_DEV

        barrier_sem = pltpu.get_barrier_semaphore()
        for nbr in (left, right):
            pltpu.semaphore_signal(
                barrier_sem, inc=1,
                device_id=(nbr,), device_id_type=pl.DeviceIdType.MESH,
            )
        pltpu.semaphore_wait(barrier_sem, 2)

        def partial(c):
            x_slice = x_ref[pl.ds(pl.multiple_of(c * M_PER, M_PER), M_PER), :]
            return jnp.dot(x_slice, w_ref[...],
                           preferred_element_type=jnp.float32)

        def make_rdma(slot_unused=None):
            return pltpu.make_async_remote_copy(
                src_ref=send_buf,
                dst_ref=recv_buf,
                send_sem=send_sem,
                recv_sem=recv_sem,
                device_id=(right,),
                device_id_type=pl.DeviceIdType.MESH,
            )

        prev = None
        for s in range(N_DEV - 1):
            c = (d - s - 1) % N_DEV
            if s == 0:
                send_buf[...] = partial(c)
            else:
                prev.wait_send()
                prev.wait_recv()
                send_buf[...] = recv_buf[...] + partial(c)
                pltpu.semaphore_signal(
                    credit_sem, inc=1,
                    device_id=(left,), device_id_type=pl.DeviceIdType.MESH,
                )
                pltpu.semaphore_wait(credit_sem, 1)
            rdma = make_rdma()
            rdma.start()
            prev = rdma

        prev.wait_send()
        prev.wait_recv()
        y = recv_buf[...] + partial(d)
        z = jnp.clip(y, -60.0, 60.0)
        out_ref[...] = y * (1.0 / (1.0 + jnp.exp(-z)))

    return pl.pallas_call(
        body,
        out_shape=jax.ShapeDtypeStruct((M_PER, n), jnp.float32),
        in_specs=[
            pl.BlockSpec(memory_space=pltpu.VMEM),
            pl.BlockSpec(memory_space=pltpu.VMEM),
        ],
        out_specs=pl.BlockSpec(memory_space=pltpu.VMEM),
        scratch_shapes=[
            pltpu.VMEM((M_PER, n), jnp.float32),
            pltpu.VMEM((M_PER, n), jnp.float32),
            pltpu.SemaphoreType.DMA,
            pltpu.SemaphoreType.DMA,
            pltpu.SemaphoreType.REGULAR,
        ],
        compiler_params=pltpu.CompilerParams(collective_id=0),
    )(x, w_mat)


# baseline (device time: 1494374 ns/iter reference)
import functools

import jax
import jax.numpy as jnp
from jax import lax
from jax.experimental import pallas as pl
from jax.experimental.pallas import tpu as pltpu

N_DEV = 16
M_PER = 512


def kernel(x, w_mat):
    k_glob, k_per = x.shape
    _, n = w_mat.shape

    def body(x_ref, w_ref, out_ref, send_buf, recv_buf,
             send_sem, recv_sem, credit_sem):
        d = lax.axis_index("i")
        left = (d - 1) % N_DEV
        right = (d + 1) % N_DEV

        barrier_sem = pltpu.get_barrier_semaphore()
        for nbr in (left, right):
            pltpu.semaphore_signal(
                barrier_sem, inc=1,
                device_id=(nbr,), device_id_type=pl.DeviceIdType.MESH,
            )
        pltpu.semaphore_wait(barrier_sem, 2)

        def partial(c):
            x_slice = x_ref[pl.ds(pl.multiple_of(c * M_PER, M_PER), M_PER), :]
            return jnp.dot(x_slice, w_ref[...],
                           preferred_element_type=jnp.float32)

        def make_rdma(slot_unused=None):
            return pltpu.make_async_remote_copy(
                src_ref=send_buf,
                dst_ref=recv_buf,
                send_sem=send_sem,
                recv_sem=recv_sem,
                device_id=(right,),
                device_id_type=pl.DeviceIdType.MESH,
            )

        prev = None
        for s in range(N_DEV - 1):
            c = (d - s - 1) % N_DEV
            if s == 0:
                send_buf[...] = partial(c)
            else:
                prev.wait_send()
                prev.wait_recv()
                send_buf[...] = recv_buf[...] + partial(c)
                pltpu.semaphore_signal(
                    credit_sem, inc=1,
                    device_id=(left,), device_id_type=pl.DeviceIdType.MESH,
                )
                pltpu.semaphore_wait(credit_sem, 1)
            rdma = make_rdma()
            rdma.start()
            prev = rdma

        prev.wait_send()
        prev.wait_recv()
        y = recv_buf[...] + partial(d)
        z = jnp.clip(y, -60.0, 60.0)
        out_ref[...] = y * (1.0 / (1.0 + jnp.exp(-z)))

    return pl.pallas_call(
        body,
        out_shape=jax.ShapeDtypeStruct((M_PER, n), jnp.float32),
        in_specs=[
            pl.BlockSpec(memory_space=pltpu.VMEM),
            pl.BlockSpec(memory_space=pltpu.VMEM),
        ],
        out_specs=pl.BlockSpec(memory_space=pltpu.VMEM),
        scratch_shapes=[
            pltpu.VMEM((M_PER, n), jnp.float32),
            pltpu.VMEM((M_PER, n), jnp.float32),
            pltpu.SemaphoreType.DMA,
            pltpu.SemaphoreType.DMA,
            pltpu.SemaphoreType.REGULAR,
        ],
        compiler_params=pltpu.CompilerParams(
            collective_id=0,
            vmem_limit_bytes=100 * 1024 * 1024,
        ),
    )(x, w_mat)


# device time: 783200 ns/iter; 1.9080x vs baseline; 1.9080x over previous
import jax
import jax.numpy as jnp
from jax import lax
from jax.experimental import pallas as pl
from jax.experimental.pallas import tpu as pltpu

N_DEV = 16
M_PER = 512


def kernel(x, w_mat):
    k_glob, k_per = x.shape
    _, n = w_mat.shape
    nh = n // 2

    def body(x_ref, w_ref, out_ref,
             send_r, recv_r, send_l, recv_l,
             send_sem_r, recv_sem_r, send_sem_l, recv_sem_l,
             credit_r, credit_l):
        d = lax.axis_index("i")
        left = (d - 1) % N_DEV
        right = (d + 1) % N_DEV

        barrier_sem = pltpu.get_barrier_semaphore()
        for nbr in (left, right):
            pltpu.semaphore_signal(
                barrier_sem, inc=1,
                device_id=(nbr,), device_id_type=pl.DeviceIdType.MESH,
            )
        pltpu.semaphore_wait(barrier_sem, 2)

        def partial(c, half):
            x_slice = x_ref[pl.ds(pl.multiple_of(c * M_PER, M_PER), M_PER), :]
            w_slice = w_ref[:, pl.ds(half * nh, nh)]
            return jnp.dot(x_slice, w_slice,
                           preferred_element_type=jnp.float32)

        def make_rdma_r():
            return pltpu.make_async_remote_copy(
                src_ref=send_r, dst_ref=recv_r,
                send_sem=send_sem_r, recv_sem=recv_sem_r,
                device_id=(right,), device_id_type=pl.DeviceIdType.MESH,
            )

        def make_rdma_l():
            return pltpu.make_async_remote_copy(
                src_ref=send_l, dst_ref=recv_l,
                send_sem=send_sem_l, recv_sem=recv_sem_l,
                device_id=(left,), device_id_type=pl.DeviceIdType.MESH,
            )

        send_r[...] = partial((d - 1) % N_DEV, 0)
        send_l[...] = partial((d + 1) % N_DEV, 1)
        prev_r = make_rdma_r()
        prev_l = make_rdma_l()
        prev_r.start()
        prev_l.start()

        for s in range(1, N_DEV - 1):
            c_r = (d - s - 1) % N_DEV
            c_l = (d + s + 1) % N_DEV
            p_r = partial(c_r, 0)
            p_l = partial(c_l, 1)

            prev_r.wait_send()
            prev_r.wait_recv()
            send_r[...] = recv_r[...] + p_r
            pltpu.semaphore_signal(
                credit_r, inc=1,
                device_id=(left,), device_id_type=pl.DeviceIdType.MESH,
            )
            prev_l.wait_send()
            prev_l.wait_recv()
            send_l[...] = recv_l[...] + p_l
            pltpu.semaphore_signal(
                credit_l, inc=1,
                device_id=(right,), device_id_type=pl.DeviceIdType.MESH,
            )
            pltpu.semaphore_wait(credit_r, 1)
            pltpu.semaphore_wait(credit_l, 1)
            prev_r = make_rdma_r()
            prev_l = make_rdma_l()
            prev_r.start()
            prev_l.start()

        p_r = partial(d, 0)
        p_l = partial(d, 1)

        def silu(y):
            z = jnp.clip(y, -60.0, 60.0)
            return y * (1.0 / (1.0 + jnp.exp(-z)))

        prev_r.wait_send()
        prev_r.wait_recv()
        out_ref[:, pl.ds(0, nh)] = silu(recv_r[...] + p_r)
        prev_l.wait_send()
        prev_l.wait_recv()
        out_ref[:, pl.ds(nh, nh)] = silu(recv_l[...] + p_l)

    return pl.pallas_call(
        body,
        out_shape=jax.ShapeDtypeStruct((M_PER, n), jnp.float32),
        in_specs=[
            pl.BlockSpec(memory_space=pltpu.VMEM),
            pl.BlockSpec(memory_space=pltpu.VMEM),
        ],
        out_specs=pl.BlockSpec(memory_space=pltpu.VMEM),
        scratch_shapes=[
            pltpu.VMEM((M_PER, nh), jnp.float32),
            pltpu.VMEM((M_PER, nh), jnp.float32),
            pltpu.VMEM((M_PER, nh), jnp.float32),
            pltpu.VMEM((M_PER, nh), jnp.float32),
            pltpu.SemaphoreType.DMA,
            pltpu.SemaphoreType.DMA,
            pltpu.SemaphoreType.DMA,
            pltpu.SemaphoreType.DMA,
            pltpu.SemaphoreType.REGULAR,
            pltpu.SemaphoreType.REGULAR,
        ],
        compiler_params=pltpu.CompilerParams(
            collective_id=0,
            vmem_limit_bytes=100 * 1024 * 1024,
        ),
    )(x, w_mat)


# device time: 705024 ns/iter; 2.1196x vs baseline; 1.1109x over previous
import jax
import jax.numpy as jnp
from jax import lax
from jax.experimental import pallas as pl
from jax.experimental.pallas import tpu as pltpu

N_DEV = 16
M_PER = 512
N_RING = 4


def kernel(x, w_mat):
    k_glob, k_per = x.shape
    _, n = w_mat.shape
    nq = n // N_RING

    def body(x_ref, w_ref, out_ref,
             send_bufs, recv_bufs, send_sems, recv_sems, credits):
        d = lax.axis_index("i")
        left = (d - 1) % N_DEV
        right = (d + 1) % N_DEV

        barrier_sem = pltpu.get_barrier_semaphore()
        for nbr in (left, right):
            pltpu.semaphore_signal(
                barrier_sem, inc=1,
                device_id=(nbr,), device_id_type=pl.DeviceIdType.MESH,
            )
        pltpu.semaphore_wait(barrier_sem, 2)

        def partial(c, r):
            x_slice = x_ref[pl.ds(pl.multiple_of(c * M_PER, M_PER), M_PER), :]
            w_slice = w_ref[:, pl.ds(r * nq, nq)]
            return jnp.dot(x_slice, w_slice,
                           preferred_element_type=jnp.float32)

        def chunk(r, s):
            return (d - s - 1) % N_DEV if r < 2 else (d + s + 1) % N_DEV

        def downstream(r):
            return right if r < 2 else left

        def upstream(r):
            return left if r < 2 else right

        def mk(r, slot):
            return pltpu.make_async_remote_copy(
                src_ref=send_bufs.at[r],
                dst_ref=recv_bufs.at[r, slot],
                send_sem=send_sems.at[r],
                recv_sem=recv_sems.at[r, slot],
                device_id=(downstream(r),),
                device_id_type=pl.DeviceIdType.MESH,
            )

        for r in range(N_RING):
            send_bufs[r, :, :] = partial(chunk(r, 0), r)
        descs = []
        for r in range(N_RING):
            rd = mk(r, 0)
            rd.start()
            descs.append(rd)

        for s in range(1, N_DEV - 1):
            ps = [partial(chunk(r, s), r) for r in range(N_RING)]
            for r in range(N_RING):
                prev = descs[r]
                prev.wait_send()
                prev.wait_recv()
                send_bufs[r, :, :] = recv_bufs[(r, (s - 1) % 2)] + ps[r]
                if s <= N_DEV - 3:
                    pltpu.semaphore_signal(
                        credits.at[r], inc=1,
                        device_id=(upstream(r),),
                        device_id_type=pl.DeviceIdType.MESH,
                    )
                if s >= 2:
                    pltpu.semaphore_wait(credits.at[r], 1)
                rd = mk(r, s % 2)
                rd.start()
                descs[r] = rd

        po = [partial(d, r) for r in range(N_RING)]

        def silu(y):
            z = jnp.clip(y, -60.0, 60.0)
            return y * (1.0 / (1.0 + jnp.exp(-z)))

        for r in range(N_RING):
            descs[r].wait_send()
            descs[r].wait_recv()
            y = recv_bufs[(r, (N_DEV - 2) % 2)] + po[r]
            out_ref[:, pl.ds(r * nq, nq)] = silu(y)

    return pl.pallas_call(
        body,
        out_shape=jax.ShapeDtypeStruct((M_PER, n), jnp.float32),
        in_specs=[
            pl.BlockSpec(memory_space=pltpu.VMEM),
            pl.BlockSpec(memory_space=pltpu.VMEM),
        ],
        out_specs=pl.BlockSpec(memory_space=pltpu.VMEM),
        scratch_shapes=[
            pltpu.VMEM((N_RING, M_PER, nq), jnp.float32),
            pltpu.VMEM((N_RING, 2, M_PER, nq), jnp.float32),
            pltpu.SemaphoreType.DMA((N_RING,)),
            pltpu.SemaphoreType.DMA((N_RING, 2)),
            pltpu.SemaphoreType.REGULAR((N_RING,)),
        ],
        compiler_params=pltpu.CompilerParams(
            collective_id=0,
            vmem_limit_bytes=100 * 1024 * 1024,
        ),
    )(x, w_mat)


# device time: 702982 ns/iter; 2.1258x vs baseline; 1.0029x over previous
import jax
import jax.numpy as jnp
from jax import lax
from jax.experimental import pallas as pl
from jax.experimental.pallas import tpu as pltpu

N_DEV = 16
M_PER = 512
N_RING = 4

_ORDER = (0, 2, 1, 3)


def kernel(x, w_mat):
    k_glob, k_per = x.shape
    _, n = w_mat.shape
    nq = n // N_RING

    def body(x_ref, w_ref, out_ref,
             send_bufs, recv_bufs, send_sems, recv_sems, credits):
        d = lax.axis_index("i")
        left = (d - 1) % N_DEV
        right = (d + 1) % N_DEV

        barrier_sem = pltpu.get_barrier_semaphore()
        for nbr in (left, right):
            pltpu.semaphore_signal(
                barrier_sem, inc=1,
                device_id=(nbr,), device_id_type=pl.DeviceIdType.MESH,
            )
        pltpu.semaphore_wait(barrier_sem, 2)

        def partial(c, r):
            x_slice = x_ref[pl.ds(pl.multiple_of(c * M_PER, M_PER), M_PER), :]
            w_slice = w_ref[:, pl.ds(r * nq, nq)]
            return jnp.dot(x_slice, w_slice,
                           preferred_element_type=jnp.float32)

        def chunk(r, s):
            return (d - s - 1) % N_DEV if r < 2 else (d + s + 1) % N_DEV

        def downstream(r):
            return right if r < 2 else left

        def upstream(r):
            return left if r < 2 else right

        def mk(r, slot):
            return pltpu.make_async_remote_copy(
                src_ref=send_bufs.at[r],
                dst_ref=recv_bufs.at[r, slot],
                send_sem=send_sems.at[r],
                recv_sem=recv_sems.at[r, slot],
                device_id=(downstream(r),),
                device_id_type=pl.DeviceIdType.MESH,
            )

        prev = [None] * N_RING
        for r in _ORDER:
            send_bufs[r, :, :] = partial(chunk(r, 0), r)
            rd = mk(r, 0)
            rd.start()
            prev[r] = rd

        for s in range(1, N_DEV - 1):
            slot = s % 2
            for r in _ORDER:
                p = partial(chunk(r, s), r)
                prev[r].wait_send()
                prev[r].wait_recv()
                send_bufs[r, :, :] = recv_bufs[(r, 1 - slot)] + p
                if s <= N_DEV - 3:
                    pltpu.semaphore_signal(
                        credits.at[r], inc=1,
                        device_id=(upstream(r),),
                        device_id_type=pl.DeviceIdType.MESH,
                    )
                if s >= 2:
                    pltpu.semaphore_wait(credits.at[r], 1)
                rd = mk(r, slot)
                rd.start()
                prev[r] = rd

        def silu(y):
            z = jnp.clip(y, -60.0, 60.0)
            return y * (1.0 / (1.0 + jnp.exp(-z)))

        for r in _ORDER:
            p = partial(d, r)
            prev[r].wait_send()
            prev[r].wait_recv()
            y = recv_bufs[(r, (N_DEV - 2) % 2)] + p
            out_ref[:, pl.ds(r * nq, nq)] = silu(y)

    return pl.pallas_call(
        body,
        out_shape=jax.ShapeDtypeStruct((M_PER, n), jnp.float32),
        in_specs=[
            pl.BlockSpec(memory_space=pltpu.VMEM),
            pl.BlockSpec(memory_space=pltpu.VMEM),
        ],
        out_specs=pl.BlockSpec(memory_space=pltpu.VMEM),
        scratch_shapes=[
            pltpu.VMEM((N_RING, M_PER, nq), jnp.float32),
            pltpu.VMEM((N_RING, 2, M_PER, nq), jnp.float32),
            pltpu.SemaphoreType.DMA((N_RING,)),
            pltpu.SemaphoreType.DMA((N_RING, 2)),
            pltpu.SemaphoreType.REGULAR((N_RING,)),
        ],
        compiler_params=pltpu.CompilerParams(
            collective_id=0,
            vmem_limit_bytes=100 * 1024 * 1024,
        ),
    )(x, w_mat)


# device time: 699872 ns/iter; 2.1352x vs baseline; 1.0044x over previous
import jax
import jax.numpy as jnp
from jax import lax
from jax.experimental import pallas as pl
from jax.experimental.pallas import tpu as pltpu

N_DEV = 16
M_PER = 512
N_RING = 4
N_STAGE = 3

_ORDER = (0, 2, 1, 3)


def kernel(x, w_mat):
    k_glob, k_per = x.shape
    _, n = w_mat.shape
    nq = n // N_RING

    def body(x_ref, w_ref, out_ref,
             x_stage, send_bufs, recv_bufs,
             stage_sems, send_sems, recv_sems, credits):
        d = lax.axis_index("i")
        left = (d - 1) % N_DEV
        right = (d + 1) % N_DEV

        barrier_sem = pltpu.get_barrier_semaphore()
        for nbr in (left, right):
            pltpu.semaphore_signal(
                barrier_sem, inc=1,
                device_id=(nbr,), device_id_type=pl.DeviceIdType.MESH,
            )
        pltpu.semaphore_wait(barrier_sem, 2)

        def chunk_dir(dir_, s):
            return (d - s - 1) % N_DEV if dir_ == 0 else (d + s + 1) % N_DEV

        stage_descs = {}

        def issue_stage(s):
            slot = s % N_STAGE
            for dir_ in range(2):
                c = chunk_dir(dir_, s)
                dsc = pltpu.make_async_copy(
                    x_ref.at[pl.ds(pl.multiple_of(c * M_PER, M_PER), M_PER), :],
                    x_stage.at[dir_, slot],
                    stage_sems.at[dir_, slot],
                )
                dsc.start()
                stage_descs[(dir_, s)] = dsc

        def wait_stage(s):
            for dir_ in range(2):
                stage_descs.pop((dir_, s)).wait()

        def partial(r, s):
            dir_ = 0 if r < 2 else 1
            x_slice = x_stage[dir_, s % N_STAGE]
            w_slice = w_ref[:, pl.ds(r * nq, nq)]
            return jnp.dot(x_slice, w_slice,
                           preferred_element_type=jnp.float32)

        def downstream(r):
            return right if r < 2 else left

        def upstream(r):
            return left if r < 2 else right

        def mk(r, slot):
            return pltpu.make_async_remote_copy(
                src_ref=send_bufs.at[r, slot],
                dst_ref=recv_bufs.at[r, slot],
                send_sem=send_sems.at[r, slot],
                recv_sem=recv_sems.at[r, slot],
                device_id=(downstream(r),),
                device_id_type=pl.DeviceIdType.MESH,
            )

        for s0 in range(N_STAGE):
            issue_stage(s0)

        prev = [None] * N_RING
        prev2 = [None] * N_RING
        wait_stage(0)
        for r in _ORDER:
            send_bufs[r, 0, :, :] = partial(r, 0)
            rd = mk(r, 0)
            rd.start()
            prev[r] = rd

        for s in range(1, N_DEV - 1):
            slot = s % 2
            if s + 2 <= N_DEV - 1:
                issue_stage(s + 2)
            wait_stage(s)
            for r in _ORDER:
                p = partial(r, s)
                if s >= 2:
                    prev2[r].wait_send()
                prev[r].wait_recv()
                send_bufs[r, slot, :, :] = recv_bufs[(r, 1 - slot)] + p
                if s <= N_DEV - 3:
                    pltpu.semaphore_signal(
                        credits.at[r], inc=1,
                        device_id=(upstream(r),),
                        device_id_type=pl.DeviceIdType.MESH,
                    )
                if s >= 2:
                    pltpu.semaphore_wait(credits.at[r], 1)
                rd = mk(r, slot)
                rd.start()
                prev2[r] = prev[r]
                prev[r] = rd

        def silu(y):
            z = jnp.clip(y, -60.0, 60.0)
            return y * (1.0 / (1.0 + jnp.exp(-z)))

        wait_stage(N_DEV - 1)
        for r in _ORDER:
            p = partial(r, N_DEV - 1)
            prev2[r].wait_send()
            prev[r].wait_send()
            prev[r].wait_recv()
            y = recv_bufs[(r, (N_DEV - 2) % 2)] + p
            out_ref[:, pl.ds(r * nq, nq)] = silu(y)

    return pl.pallas_call(
        body,
        out_shape=jax.ShapeDtypeStruct((M_PER, n), jnp.float32),
        in_specs=[
            pl.BlockSpec(memory_space=pl.ANY),
            pl.BlockSpec(memory_space=pltpu.VMEM),
        ],
        out_specs=pl.BlockSpec(memory_space=pltpu.VMEM),
        scratch_shapes=[
            pltpu.VMEM((2, N_STAGE, M_PER, k_per), jnp.float32),
            pltpu.VMEM((N_RING, 2, M_PER, nq), jnp.float32),
            pltpu.VMEM((N_RING, 2, M_PER, nq), jnp.float32),
            pltpu.SemaphoreType.DMA((2, N_STAGE)),
            pltpu.SemaphoreType.DMA((N_RING, 2)),
            pltpu.SemaphoreType.DMA((N_RING, 2)),
            pltpu.SemaphoreType.REGULAR((N_RING,)),
        ],
        compiler_params=pltpu.CompilerParams(
            collective_id=0,
            vmem_limit_bytes=100 * 1024 * 1024,
        ),
    )(x, w_mat)


# device time: 698142 ns/iter; 2.1405x vs baseline; 1.0025x over previous
import jax
import jax.numpy as jnp
from jax import lax
from jax.experimental import pallas as pl
from jax.experimental.pallas import tpu as pltpu

N_DEV = 16
M_PER = 512
N_RING = 4
N_STAGE = 3

_ORDER = (0, 2, 1, 3)


def kernel(x, w_mat):
    k_glob, k_per = x.shape
    _, n = w_mat.shape
    nq = n // N_RING

    def body(x_ref, w_ref, out_ref,
             x_stage, send_bufs, recv_bufs,
             stage_sems, send_sems, recv_sems, credits):
        d = lax.axis_index("i")
        left = (d - 1) % N_DEV
        right = (d + 1) % N_DEV

        barrier_sem = pltpu.get_barrier_semaphore()
        for nbr in (left, right):
            pltpu.semaphore_signal(
                barrier_sem, inc=1,
                device_id=(nbr,), device_id_type=pl.DeviceIdType.MESH,
            )
        pltpu.semaphore_wait(barrier_sem, 2)

        def chunk_dir(dir_, s):
            return (d - s - 1) % N_DEV if dir_ == 0 else (d + s + 1) % N_DEV

        stage_descs = {}

        def issue_stage(s):
            slot = s % N_STAGE
            for dir_ in range(2):
                c = chunk_dir(dir_, s)
                dsc = pltpu.make_async_copy(
                    x_ref.at[pl.ds(pl.multiple_of(c * M_PER, M_PER), M_PER), :],
                    x_stage.at[dir_, slot],
                    stage_sems.at[dir_, slot],
                )
                dsc.start()
                stage_descs[(dir_, s)] = dsc

        def wait_stage(s):
            for dir_ in range(2):
                stage_descs.pop((dir_, s)).wait()

        def partial(r, s):
            dir_ = 0 if r < 2 else 1
            x_slice = x_stage[dir_, s % N_STAGE]
            w_slice = w_ref[:, pl.ds(r * nq, nq)]
            return jnp.dot(x_slice, w_slice,
                           preferred_element_type=jnp.float32)

        def downstream(r):
            return right if r < 2 else left

        def upstream(r):
            return left if r < 2 else right

        def mk(r, slot):
            return pltpu.make_async_remote_copy(
                src_ref=send_bufs.at[r, slot],
                dst_ref=recv_bufs.at[r, slot],
                send_sem=send_sems.at[r, slot],
                recv_sem=recv_sems.at[r, slot],
                device_id=(downstream(r),),
                device_id_type=pl.DeviceIdType.MESH,
            )

        for s0 in range(N_STAGE):
            issue_stage(s0)

        prev = [None] * N_RING
        prev2 = [None] * N_RING
        wait_stage(0)
        for r in _ORDER:
            send_bufs[r, 0, :, :] = partial(r, 0)
            rd = mk(r, 0)
            rd.start()
            prev[r] = rd

        for s in range(1, N_DEV - 1):
            slot = s % 2
            if s + 2 <= N_DEV - 1:
                issue_stage(s + 2)
            wait_stage(s)
            for r in _ORDER:
                p = partial(r, s)
                if s >= 2:
                    prev2[r].wait_send()
                prev[r].wait_recv()
                send_bufs[r, slot, :, :] = recv_bufs[(r, 1 - slot)] + p
                if s <= N_DEV - 3:
                    pltpu.semaphore_signal(
                        credits.at[r], inc=1,
                        device_id=(upstream(r),),
                        device_id_type=pl.DeviceIdType.MESH,
                    )
                if s >= 2:
                    pltpu.semaphore_wait(credits.at[r], 1)
                rd = mk(r, slot)
                rd.start()
                prev2[r] = prev[r]
                prev[r] = rd

        def silu(y):
            z = jnp.clip(y, -60.0, 60.0)
            return y * (1.0 / (1.0 + jnp.exp(-z)))

        wait_stage(N_DEV - 1)
        out_descs = []
        for r in _ORDER:
            p = partial(r, N_DEV - 1)
            prev2[r].wait_send()
            prev[r].wait_send()
            prev[r].wait_recv()
            y = recv_bufs[(r, (N_DEV - 2) % 2)] + p
            send_bufs[r, 1, :, :] = silu(y)
            dsc = pltpu.make_async_copy(
                send_bufs.at[r, 1],
                out_ref.at[:, pl.ds(r * nq, nq)],
                stage_sems.at[0 if r < 2 else 1, r % 2],
            )
            dsc.start()
            out_descs.append(dsc)
        for dsc in out_descs:
            dsc.wait()

    return pl.pallas_call(
        body,
        out_shape=jax.ShapeDtypeStruct((M_PER, n), jnp.float32),
        in_specs=[
            pl.BlockSpec(memory_space=pl.ANY),
            pl.BlockSpec(memory_space=pltpu.VMEM),
        ],
        out_specs=pl.BlockSpec(memory_space=pl.ANY),
        scratch_shapes=[
            pltpu.VMEM((2, N_STAGE, M_PER, k_per), jnp.float32),
            pltpu.VMEM((N_RING, 2, M_PER, nq), jnp.float32),
            pltpu.VMEM((N_RING, 2, M_PER, nq), jnp.float32),
            pltpu.SemaphoreType.DMA((2, N_STAGE)),
            pltpu.SemaphoreType.DMA((N_RING, 2)),
            pltpu.SemaphoreType.DMA((N_RING, 2)),
            pltpu.SemaphoreType.REGULAR((N_RING,)),
        ],
        compiler_params=pltpu.CompilerParams(
            collective_id=0,
            vmem_limit_bytes=100 * 1024 * 1024,
        ),
    )(x, w_mat)
